# baseline (speedup 1.0000x reference)
import jax
import jax.numpy as jnp
from jax.experimental import pallas as pl
from jax.experimental.pallas import tpu as pltpu


def _mlp_head_body(pooled, w0, b0, w1, b1, w2, b2, w3, b3, out):
    h = jnp.maximum(pooled[...] @ w0[...] + b0[...], 0.0)
    h = jnp.maximum(h @ w1[...] + b1[...], 0.0)
    h = jnp.maximum(h @ w2[...] + b2[...], 0.0)
    out[...] = h @ w3[...] + b3[...]


def _mlp_head(pooled, p):
    B = pooled.shape[0]
    return pl.pallas_call(
        _mlp_head_body,
        out_shape=jax.ShapeDtypeStruct((B, 1), jnp.float32),
    )(pooled, p['fcxd_W'], p['fcxd_b'][None, :], p['fc1_W'], p['fc1_b'][None, :],
      p['fc2_W'], p['fc2_b'][None, :], p['out_W'], p['out_b'][None, :])


def _gcn(x, src, dst, ew, W, b):
    n = x.shape[0]
    loop = jnp.arange(n)
    s = jnp.concatenate([src, loop])
    d = jnp.concatenate([dst, loop])
    w = jnp.concatenate([ew, jnp.ones((n,), ew.dtype)])
    deg = jax.ops.segment_sum(w, d, num_segments=n)
    dis = jnp.where(deg > 0, deg ** -0.5, 0.0)
    norm = dis[s] * w * dis[d]
    xw = x @ W
    out = jax.ops.segment_sum(xw[s] * norm[:, None], d, num_segments=n)
    return out + b


def _gat(x, src, dst, W, a_src, a_dst, b):
    n = x.shape[0]
    loop = jnp.arange(n)
    s = jnp.concatenate([src, loop])
    d = jnp.concatenate([dst, loop])
    xw = x @ W
    al_s = xw @ a_src
    al_d = xw @ a_dst
    e = jax.nn.leaky_relu(al_s[s] + al_d[d], 0.2)
    m = jax.ops.segment_max(e, d, num_segments=n)
    m = jnp.where(jnp.isfinite(m), m, 0.0)
    ex = jnp.exp(e - m[d])
    den = jax.ops.segment_sum(ex, d, num_segments=n)
    alpha = ex / (den[d] + 1e-16)
    out = jax.ops.segment_sum(xw[s] * alpha[:, None], d, num_segments=n)
    return out + b


def _bn(h, g, be):
    mu = h.mean(axis=0)
    var = h.var(axis=0)
    return (h - mu) / jnp.sqrt(var + 1e-5) * g + be


def kernel(x, edge_index, edge_weight, batch, params):
    src = edge_index[0]
    dst = edge_index[1]
    ew = edge_weight.mean(axis=1)
    h = jax.nn.relu(_gcn(x, src, dst, ew, params['W0'], params['b0']))
    h = _bn(h, params['g0'], params['be0'])
    for i in range(1, 6):
        h = jax.nn.relu(_gat(h, src, dst, params['W%d' % i], params['as%d' % i],
                             params['ad%d' % i], params['bb%d' % i]))
        h = _bn(h, params['g%d' % i], params['be%d' % i])
    pooled = jax.ops.segment_sum(h, batch, num_segments=256)
    return _mlp_head(pooled, params)
